# baseline (speedup 1.0000x reference)
import functools

import jax
import jax.numpy as jnp
from jax import lax
from jax.experimental import pallas as pl
from jax.experimental.pallas import tpu as pltpu
from jax.experimental.pallas import tpu_sc as plsc

EPS = 1e-5
HID = 16


def _sc_gather_rows(h, src):
    n_nodes, d = h.shape
    n_edges = src.shape[0]
    info = plsc.get_sparse_core_info()
    nw = info.num_cores * info.num_subcores
    per_w = n_edges // nw
    ch = 2000
    n_ch = per_w // ch
    assert per_w % ch == 0 and n_edges % nw == 0

    mesh = plsc.VectorSubcoreMesh(core_axis_name="c", subcore_axis_name="s")

    @functools.partial(
        pl.kernel,
        out_type=jax.ShapeDtypeStruct((n_edges, d), jnp.float32),
        mesh=mesh,
        compiler_params=pltpu.CompilerParams(use_tc_tiling_on_sc=False),
        scratch_types=[
            pltpu.VMEM((ch,), jnp.int32),
            pltpu.VMEM((ch, d), jnp.float32),
            pltpu.SemaphoreType.DMA,
        ],
    )
    def k(h_hbm, src_hbm, out_hbm, idx_v, rows_v, sem):
        wid = lax.axis_index("s") * info.num_cores + lax.axis_index("c")
        base = wid * per_w

        def body(i, _):
            off = base + i * ch
            pltpu.sync_copy(src_hbm.at[pl.ds(off, ch)], idx_v)
            pltpu.async_copy(h_hbm.at[idx_v], rows_v, sem).wait()
            pltpu.sync_copy(rows_v, out_hbm.at[pl.ds(off, ch)])
            return 0

        lax.fori_loop(0, n_ch, body, 0)

    return k(h, src)


def _sc_scatter_add(m, dst, n_nodes):
    n_edges, d = m.shape
    info = plsc.get_sparse_core_info()
    nc, ns = info.num_cores, info.num_subcores
    nw = nc * ns
    per_w = n_edges // nw
    ch = 2000
    n_ch = per_w // ch
    rows_per_tile = n_nodes // ns
    assert n_nodes % ns == 0 and per_w % ch == 0

    zeros = jnp.zeros((n_nodes, d), jnp.float32)
    mesh = plsc.VectorSubcoreMesh(core_axis_name="c", subcore_axis_name="s")

    @functools.partial(
        pl.kernel,
        out_type=jax.ShapeDtypeStruct((nc, n_nodes, d), jnp.float32),
        mesh=mesh,
        compiler_params=pltpu.CompilerParams(use_tc_tiling_on_sc=False),
        scratch_types=[
            pltpu.VMEM((ch,), jnp.int32),
            pltpu.VMEM((ch, d), jnp.float32),
            pltpu.VMEM_SHARED((n_nodes, d), jnp.float32),
        ],
    )
    def k(m_hbm, dst_hbm, z_hbm, out_hbm, idx_v, rows_v, agg_sh):
        cid = lax.axis_index("c")
        sid = lax.axis_index("s")
        stripe = sid * rows_per_tile
        pltpu.sync_copy(z_hbm.at[pl.ds(stripe, rows_per_tile)],
                        agg_sh.at[pl.ds(stripe, rows_per_tile)])
        plsc.subcore_barrier()

        wid = sid * nc + cid
        base = wid * per_w

        def body(i, _):
            off = base + i * ch
            pltpu.sync_copy(dst_hbm.at[pl.ds(off, ch)], idx_v)
            pltpu.sync_copy(m_hbm.at[pl.ds(off, ch)], rows_v)
            pltpu.sync_copy(rows_v, agg_sh.at[idx_v], add=True)
            return 0

        lax.fori_loop(0, n_ch, body, 0)
        plsc.subcore_barrier()
        pltpu.sync_copy(agg_sh.at[pl.ds(stripe, rows_per_tile)],
                        out_hbm.at[cid, pl.ds(stripe, rows_per_tile)])

    return k(m, dst, zeros)


def _matmul_bias(x, w, b):
    n, _ = x.shape
    dout = w.shape[1]

    def body(x_ref, w_ref, b_ref, o_ref):
        o_ref[...] = (
            jnp.dot(x_ref[...], w_ref[...], preferred_element_type=jnp.float32)
            + b_ref[...]
        )

    return pl.pallas_call(
        body, out_shape=jax.ShapeDtypeStruct((n, dout), jnp.float32)
    )(x, w, b.reshape(1, dout))


def _edge_messages(edge_attr, hsrc, mlp_w, mlp_b):
    n_edges, ed = edge_attr.shape
    hh = HID * HID
    blk = 8000
    grid = (n_edges // blk,)

    ea_p = edge_attr.reshape(n_edges // 8, 8 * ed)
    hs_p = hsrc.reshape(n_edges // 8, 8 * HID)
    eye8 = jnp.eye(8, dtype=jnp.float32)
    w_big = jnp.kron(eye8, mlp_w)
    b_big = jnp.tile(mlp_b, 8).reshape(1, 8 * hh)
    r_small = (jnp.arange(hh)[None, :] // HID
               == jnp.arange(HID)[:, None]).astype(jnp.float32)
    s_small = (jnp.arange(hh)[:, None] % HID
               == jnp.arange(HID)[None, :]).astype(jnp.float32)
    r_big = jnp.kron(eye8, r_small)
    s_big = jnp.kron(eye8, s_small)

    rblk = blk // 8

    def body(ea_ref, hs_ref, w_ref, b_ref, rep_ref, red_ref, m_ref):
        we = jnp.maximum(
            jnp.dot(ea_ref[...], w_ref[...], preferred_element_type=jnp.float32)
            + b_ref[...],
            0.0,
        )
        we = we.astype(jnp.bfloat16).astype(jnp.float32)
        rep = jnp.dot(
            hs_ref[...], rep_ref[...], preferred_element_type=jnp.float32,
            precision=jax.lax.Precision.HIGHEST,
        )
        rep = rep.astype(jnp.bfloat16).astype(jnp.float32)
        m_ref[...] = jnp.dot(
            we * rep, red_ref[...], preferred_element_type=jnp.float32,
            precision=jax.lax.Precision.HIGHEST,
        )

    m_p = pl.pallas_call(
        body,
        grid=grid,
        in_specs=[
            pl.BlockSpec((rblk, 8 * ed), lambda i: (i, 0)),
            pl.BlockSpec((rblk, 8 * HID), lambda i: (i, 0)),
            pl.BlockSpec((8 * ed, 8 * hh), lambda i: (0, 0)),
            pl.BlockSpec((1, 8 * hh), lambda i: (0, 0)),
            pl.BlockSpec((8 * HID, 8 * hh), lambda i: (0, 0)),
            pl.BlockSpec((8 * hh, 8 * HID), lambda i: (0, 0)),
        ],
        out_specs=pl.BlockSpec((rblk, 8 * HID), lambda i: (i, 0)),
        out_shape=jax.ShapeDtypeStruct((n_edges // 8, 8 * HID), jnp.float32),
    )(ea_p, hs_p, w_big, b_big, r_big, s_big)
    return m_p.reshape(n_edges, HID)


def _node_tail(part0, part1, h, lp, bp):
    n, d = h.shape
    trans = bp["final_W"].shape[1]

    def body(p0_ref, p1_ref, h_ref, rootw_ref, convb_ref, gnw_ref, gnb_ref,
             gnms_ref, tw_ref, tb_ref, fw_ref, fb_ref, gfw_ref, gfb_ref,
             gfms_ref, o_ref):
        h_ = h_ref[...]
        out = (
            p0_ref[...] + p1_ref[...]
            + jnp.dot(h_, rootw_ref[...], preferred_element_type=jnp.float32)
            + convb_ref[...]
        )
        mean = jnp.mean(out, axis=0, keepdims=True)
        cent = out - gnms_ref[...] * mean
        var = jnp.mean(cent * cent, axis=0, keepdims=True)
        gn = gnw_ref[...] * cent / jnp.sqrt(var + EPS) + gnb_ref[...]
        h_conv = jnp.maximum(gn, 0.0) + h_
        tw = tw_ref[...]
        h2 = jnp.maximum(
            jnp.dot(h_, tw[:d], preferred_element_type=jnp.float32)
            + jnp.dot(h_conv, tw[d:], preferred_element_type=jnp.float32)
            + tb_ref[...],
            0.0,
        )
        hf = (
            jnp.dot(h2, fw_ref[...], preferred_element_type=jnp.float32)
            + fb_ref[...]
        )
        mean2 = jnp.mean(hf, axis=0, keepdims=True)
        cent2 = hf - gfms_ref[...] * mean2
        var2 = jnp.mean(cent2 * cent2, axis=0, keepdims=True)
        o_ref[...] = jnp.maximum(
            gfw_ref[...] * cent2 / jnp.sqrt(var2 + EPS) + gfb_ref[...], 0.0
        )

    return pl.pallas_call(
        body, out_shape=jax.ShapeDtypeStruct((n, trans), jnp.float32)
    )(
        part0, part1, h,
        lp["root_W"], lp["conv_b"].reshape(1, d),
        lp["gn_w"].reshape(1, d), lp["gn_b"].reshape(1, d),
        lp["gn_ms"].reshape(1, d),
        lp["trans_W"], lp["trans_b"].reshape(1, d),
        bp["final_W"], bp["final_b"].reshape(1, trans),
        bp["gnf_w"].reshape(1, trans), bp["gnf_b"].reshape(1, trans),
        bp["gnf_ms"].reshape(1, trans),
    )


def _jk_combine(h1, h2, jk):
    n, _ = h1.shape
    out_d = jk["projs"][0]["W"].shape[1]

    def body(h1_ref, h2_ref, w1_ref, b1_ref, w2_ref, b2_ref, a1_ref, a2_ref,
             o_ref):
        z1 = (
            jnp.dot(h1_ref[...], w1_ref[...], preferred_element_type=jnp.float32)
            + b1_ref[...]
        )
        z2 = (
            jnp.dot(h2_ref[...], w2_ref[...], preferred_element_type=jnp.float32)
            + b2_ref[...]
        )
        s1 = jnp.dot(z1, a1_ref[...], preferred_element_type=jnp.float32)
        s2 = jnp.dot(z2, a2_ref[...], preferred_element_type=jnp.float32)
        mx = jnp.maximum(s1, s2)
        e1 = jnp.exp(s1 - mx)
        e2 = jnp.exp(s2 - mx)
        o_ref[...] = (e1 * z1 + e2 * z2) / (e1 + e2)

    return pl.pallas_call(
        body, out_shape=jax.ShapeDtypeStruct((n, out_d), jnp.float32)
    )(
        h1, h2,
        jk["projs"][0]["W"], jk["projs"][0]["b"].reshape(1, out_d),
        jk["projs"][1]["W"], jk["projs"][1]["b"].reshape(1, out_d),
        jk["attn"][0].reshape(out_d, 1), jk["attn"][1].reshape(out_d, 1),
    )


def kernel(x, edge_attr, params, edge_index):
    src = edge_index[0]
    dst = edge_index[1]
    n_nodes = x.shape[0]
    hs = []
    h_in = x
    for bp in params["blocks"]:
        h = _matmul_bias(h_in, bp["init_W"], bp["init_b"])
        for lp in bp["layers"]:
            hsrc = _sc_gather_rows(h, src)
            m = _edge_messages(edge_attr, hsrc, lp["mlp_W"], lp["mlp_b"])
            parts = _sc_scatter_add(m, dst, n_nodes)
            h = _node_tail(parts[0], parts[1], h, lp, bp)
        hs.append(h)
        h_in = h
    return _jk_combine(hs[0], hs[1], params["jk"])


# speedup vs baseline: 1.9174x; 1.9174x over previous
import functools

import jax
import jax.numpy as jnp
from jax import lax
from jax.experimental import pallas as pl
from jax.experimental.pallas import tpu as pltpu
from jax.experimental.pallas import tpu_sc as plsc

EPS = 1e-5
HID = 16


def _sc_gather_rows(h, src):
    n_nodes, d = h.shape
    n_edges = src.shape[0]
    info = plsc.get_sparse_core_info()
    nw = info.num_cores * info.num_subcores
    per_w = n_edges // nw
    ch = 2000
    n_ch = per_w // ch
    assert per_w % ch == 0 and n_edges % nw == 0

    mesh = plsc.VectorSubcoreMesh(core_axis_name="c", subcore_axis_name="s")

    @functools.partial(
        pl.kernel,
        out_type=jax.ShapeDtypeStruct((n_edges, d), jnp.float32),
        mesh=mesh,
        compiler_params=pltpu.CompilerParams(use_tc_tiling_on_sc=False),
        scratch_types=[
            pltpu.VMEM((ch,), jnp.int32),
            pltpu.VMEM((ch, d), jnp.float32),
            pltpu.SemaphoreType.DMA,
        ],
    )
    def k(h_hbm, src_hbm, out_hbm, idx_v, rows_v, sem):
        wid = lax.axis_index("s") * info.num_cores + lax.axis_index("c")
        base = wid * per_w

        def body(i, _):
            off = base + i * ch
            pltpu.sync_copy(src_hbm.at[pl.ds(off, ch)], idx_v)
            pltpu.async_copy(h_hbm.at[idx_v], rows_v, sem).wait()
            pltpu.sync_copy(rows_v, out_hbm.at[pl.ds(off, ch)])
            return 0

        lax.fori_loop(0, n_ch, body, 0)

    return k(h, src)


def _sc_scatter_add(m, dst, n_nodes):
    n_edges, d = m.shape
    info = plsc.get_sparse_core_info()
    nc, ns = info.num_cores, info.num_subcores
    nw = nc * ns
    per_w = n_edges // nw
    ch = 2000
    n_ch = per_w // ch
    rows_per_tile = n_nodes // ns
    assert n_nodes % ns == 0 and per_w % ch == 0

    zeros = jnp.zeros((n_nodes, d), jnp.float32)
    mesh = plsc.VectorSubcoreMesh(core_axis_name="c", subcore_axis_name="s")

    @functools.partial(
        pl.kernel,
        out_type=jax.ShapeDtypeStruct((nc, n_nodes, d), jnp.float32),
        mesh=mesh,
        compiler_params=pltpu.CompilerParams(use_tc_tiling_on_sc=False),
        scratch_types=[
            pltpu.VMEM((ch,), jnp.int32),
            pltpu.VMEM((ch, d), jnp.float32),
            pltpu.VMEM_SHARED((n_nodes, d), jnp.float32),
        ],
    )
    def k(m_hbm, dst_hbm, z_hbm, out_hbm, idx_v, rows_v, agg_sh):
        cid = lax.axis_index("c")
        sid = lax.axis_index("s")
        stripe = sid * rows_per_tile
        pltpu.sync_copy(z_hbm.at[pl.ds(stripe, rows_per_tile)],
                        agg_sh.at[pl.ds(stripe, rows_per_tile)])
        plsc.subcore_barrier()

        wid = sid * nc + cid
        base = wid * per_w

        def body(i, _):
            off = base + i * ch
            pltpu.sync_copy(dst_hbm.at[pl.ds(off, ch)], idx_v)
            pltpu.sync_copy(m_hbm.at[pl.ds(off, ch)], rows_v)
            pltpu.sync_copy(rows_v, agg_sh.at[idx_v], add=True)
            return 0

        lax.fori_loop(0, n_ch, body, 0)
        plsc.subcore_barrier()
        pltpu.sync_copy(agg_sh.at[pl.ds(stripe, rows_per_tile)],
                        out_hbm.at[cid, pl.ds(stripe, rows_per_tile)])

    return k(m, dst, zeros)


def _matmul_bias(x, w, b):
    n, _ = x.shape
    dout = w.shape[1]

    def body(x_ref, w_ref, b_ref, o_ref):
        o_ref[...] = (
            jnp.dot(x_ref[...], w_ref[...], preferred_element_type=jnp.float32)
            + b_ref[...]
        )

    return pl.pallas_call(
        body, out_shape=jax.ShapeDtypeStruct((n, dout), jnp.float32)
    )(x, w, b.reshape(1, dout))


def _edge_messages(edge_attr, hsrc, mlp_w, mlp_b):
    n_edges, ed = edge_attr.shape
    hh = HID * HID
    blk = 8000
    grid = (n_edges // blk,)

    ea_p = edge_attr.reshape(n_edges // 8, 8 * ed)
    hs_p = hsrc.reshape(n_edges // 8, 8 * HID)
    eye8 = jnp.eye(8, dtype=jnp.float32)
    w_big = jnp.kron(eye8, mlp_w)
    b_big = jnp.tile(mlp_b, 8).reshape(1, 8 * hh)
    r_small = (jnp.arange(hh)[None, :] // HID
               == jnp.arange(HID)[:, None]).astype(jnp.float32)
    s_small = (jnp.arange(hh)[:, None] % HID
               == jnp.arange(HID)[None, :]).astype(jnp.float32)
    r_big = jnp.kron(eye8, r_small)
    s_big = jnp.kron(eye8, s_small)

    rblk = blk // 8

    def body(ea_ref, hs_ref, w_ref, b_ref, rep_ref, red_ref, m_ref):
        we = jnp.maximum(
            jnp.dot(ea_ref[...], w_ref[...], preferred_element_type=jnp.float32)
            + b_ref[...],
            0.0,
        )
        we = we.astype(jnp.bfloat16).astype(jnp.float32)
        rep = jnp.dot(
            hs_ref[...], rep_ref[...], preferred_element_type=jnp.float32,
        )
        prod = we * rep
        p_hi = prod.astype(jnp.bfloat16).astype(jnp.float32)
        p_lo = prod - p_hi
        m_ref[...] = (
            jnp.dot(p_hi, red_ref[...], preferred_element_type=jnp.float32)
            + jnp.dot(p_lo, red_ref[...], preferred_element_type=jnp.float32)
        )

    m_p = pl.pallas_call(
        body,
        grid=grid,
        in_specs=[
            pl.BlockSpec((rblk, 8 * ed), lambda i: (i, 0)),
            pl.BlockSpec((rblk, 8 * HID), lambda i: (i, 0)),
            pl.BlockSpec((8 * ed, 8 * hh), lambda i: (0, 0)),
            pl.BlockSpec((1, 8 * hh), lambda i: (0, 0)),
            pl.BlockSpec((8 * HID, 8 * hh), lambda i: (0, 0)),
            pl.BlockSpec((8 * hh, 8 * HID), lambda i: (0, 0)),
        ],
        out_specs=pl.BlockSpec((rblk, 8 * HID), lambda i: (i, 0)),
        out_shape=jax.ShapeDtypeStruct((n_edges // 8, 8 * HID), jnp.float32),
    )(ea_p, hs_p, w_big, b_big, r_big, s_big)
    return m_p.reshape(n_edges, HID)


def _node_tail(part0, part1, h, lp, bp):
    n, d = h.shape
    trans = bp["final_W"].shape[1]

    def body(p0_ref, p1_ref, h_ref, rootw_ref, convb_ref, gnw_ref, gnb_ref,
             gnms_ref, tw_ref, tb_ref, fw_ref, fb_ref, gfw_ref, gfb_ref,
             gfms_ref, o_ref):
        h_ = h_ref[...]
        out = (
            p0_ref[...] + p1_ref[...]
            + jnp.dot(h_, rootw_ref[...], preferred_element_type=jnp.float32)
            + convb_ref[...]
        )
        mean = jnp.mean(out, axis=0, keepdims=True)
        cent = out - gnms_ref[...] * mean
        var = jnp.mean(cent * cent, axis=0, keepdims=True)
        gn = gnw_ref[...] * cent / jnp.sqrt(var + EPS) + gnb_ref[...]
        h_conv = jnp.maximum(gn, 0.0) + h_
        tw = tw_ref[...]
        h2 = jnp.maximum(
            jnp.dot(h_, tw[:d], preferred_element_type=jnp.float32)
            + jnp.dot(h_conv, tw[d:], preferred_element_type=jnp.float32)
            + tb_ref[...],
            0.0,
        )
        hf = (
            jnp.dot(h2, fw_ref[...], preferred_element_type=jnp.float32)
            + fb_ref[...]
        )
        mean2 = jnp.mean(hf, axis=0, keepdims=True)
        cent2 = hf - gfms_ref[...] * mean2
        var2 = jnp.mean(cent2 * cent2, axis=0, keepdims=True)
        o_ref[...] = jnp.maximum(
            gfw_ref[...] * cent2 / jnp.sqrt(var2 + EPS) + gfb_ref[...], 0.0
        )

    return pl.pallas_call(
        body, out_shape=jax.ShapeDtypeStruct((n, trans), jnp.float32)
    )(
        part0, part1, h,
        lp["root_W"], lp["conv_b"].reshape(1, d),
        lp["gn_w"].reshape(1, d), lp["gn_b"].reshape(1, d),
        lp["gn_ms"].reshape(1, d),
        lp["trans_W"], lp["trans_b"].reshape(1, d),
        bp["final_W"], bp["final_b"].reshape(1, trans),
        bp["gnf_w"].reshape(1, trans), bp["gnf_b"].reshape(1, trans),
        bp["gnf_ms"].reshape(1, trans),
    )


def _jk_combine(h1, h2, jk):
    n, _ = h1.shape
    out_d = jk["projs"][0]["W"].shape[1]

    def body(h1_ref, h2_ref, w1_ref, b1_ref, w2_ref, b2_ref, a1_ref, a2_ref,
             o_ref):
        z1 = (
            jnp.dot(h1_ref[...], w1_ref[...], preferred_element_type=jnp.float32)
            + b1_ref[...]
        )
        z2 = (
            jnp.dot(h2_ref[...], w2_ref[...], preferred_element_type=jnp.float32)
            + b2_ref[...]
        )
        s1 = jnp.dot(z1, a1_ref[...], preferred_element_type=jnp.float32)
        s2 = jnp.dot(z2, a2_ref[...], preferred_element_type=jnp.float32)
        mx = jnp.maximum(s1, s2)
        e1 = jnp.exp(s1 - mx)
        e2 = jnp.exp(s2 - mx)
        o_ref[...] = (e1 * z1 + e2 * z2) / (e1 + e2)

    return pl.pallas_call(
        body, out_shape=jax.ShapeDtypeStruct((n, out_d), jnp.float32)
    )(
        h1, h2,
        jk["projs"][0]["W"], jk["projs"][0]["b"].reshape(1, out_d),
        jk["projs"][1]["W"], jk["projs"][1]["b"].reshape(1, out_d),
        jk["attn"][0].reshape(out_d, 1), jk["attn"][1].reshape(out_d, 1),
    )


def kernel(x, edge_attr, params, edge_index):
    src = edge_index[0]
    dst = edge_index[1]
    n_nodes = x.shape[0]
    hs = []
    h_in = x
    for bp in params["blocks"]:
        h = _matmul_bias(h_in, bp["init_W"], bp["init_b"])
        for lp in bp["layers"]:
            hsrc = _sc_gather_rows(h, src)
            m = _edge_messages(edge_attr, hsrc, lp["mlp_W"], lp["mlp_b"])
            parts = _sc_scatter_add(m, dst, n_nodes)
            h = _node_tail(parts[0], parts[1], h, lp, bp)
        hs.append(h)
        h_in = h
    return _jk_combine(hs[0], hs[1], params["jk"])


# speedup vs baseline: 2.1022x; 1.0964x over previous
import functools

import jax
import jax.numpy as jnp
from jax import lax
from jax.experimental import pallas as pl
from jax.experimental.pallas import tpu as pltpu
from jax.experimental.pallas import tpu_sc as plsc

EPS = 1e-5
HID = 16


def _sc_gather_rows(h, src):
    n_nodes, d = h.shape
    n_edges = src.shape[0]
    info = plsc.get_sparse_core_info()
    nw = info.num_cores * info.num_subcores
    per_w = n_edges // nw
    ch = 2000
    n_ch = per_w // ch
    assert per_w % ch == 0 and n_edges % nw == 0

    mesh = plsc.VectorSubcoreMesh(core_axis_name="c", subcore_axis_name="s")

    @functools.partial(
        pl.kernel,
        out_type=jax.ShapeDtypeStruct((n_edges, d), jnp.float32),
        mesh=mesh,
        compiler_params=pltpu.CompilerParams(use_tc_tiling_on_sc=False),
        scratch_types=[
            pltpu.VMEM((ch,), jnp.int32),
            pltpu.VMEM((ch, d), jnp.float32),
            pltpu.SemaphoreType.DMA,
        ],
    )
    def k(h_hbm, src_hbm, out_hbm, idx_v, rows_v, sem):
        wid = lax.axis_index("s") * info.num_cores + lax.axis_index("c")
        base = wid * per_w

        def body(i, _):
            off = base + i * ch
            pltpu.sync_copy(src_hbm.at[pl.ds(off, ch)], idx_v)
            pltpu.async_copy(h_hbm.at[idx_v], rows_v, sem).wait()
            pltpu.sync_copy(rows_v, out_hbm.at[pl.ds(off, ch)])
            return 0

        lax.fori_loop(0, n_ch, body, 0)

    return k(h, src)


def _sc_scatter_add(m, dst, n_nodes):
    n_edges, d = m.shape
    info = plsc.get_sparse_core_info()
    nc, ns = info.num_cores, info.num_subcores
    nw = nc * ns
    per_w = n_edges // nw
    ch = 2000
    n_ch = per_w // ch
    rows_per_tile = n_nodes // ns
    assert n_nodes % ns == 0 and per_w % ch == 0

    zeros = jnp.zeros((n_nodes, d), jnp.float32)
    mesh = plsc.VectorSubcoreMesh(core_axis_name="c", subcore_axis_name="s")

    @functools.partial(
        pl.kernel,
        out_type=jax.ShapeDtypeStruct((nc, n_nodes, d), jnp.float32),
        mesh=mesh,
        compiler_params=pltpu.CompilerParams(use_tc_tiling_on_sc=False),
        scratch_types=[
            pltpu.VMEM((ch,), jnp.int32),
            pltpu.VMEM((ch, d), jnp.float32),
            pltpu.VMEM_SHARED((n_nodes, d), jnp.float32),
        ],
    )
    def k(m_hbm, dst_hbm, z_hbm, out_hbm, idx_v, rows_v, agg_sh):
        cid = lax.axis_index("c")
        sid = lax.axis_index("s")
        stripe = sid * rows_per_tile
        pltpu.sync_copy(z_hbm.at[pl.ds(stripe, rows_per_tile)],
                        agg_sh.at[pl.ds(stripe, rows_per_tile)])
        plsc.subcore_barrier()

        wid = sid * nc + cid
        base = wid * per_w

        def body(i, _):
            off = base + i * ch
            pltpu.sync_copy(dst_hbm.at[pl.ds(off, ch)], idx_v)
            pltpu.sync_copy(m_hbm.at[pl.ds(off, ch)], rows_v)
            pltpu.sync_copy(rows_v, agg_sh.at[idx_v], add=True)
            return 0

        lax.fori_loop(0, n_ch, body, 0)
        plsc.subcore_barrier()
        pltpu.sync_copy(agg_sh.at[pl.ds(stripe, rows_per_tile)],
                        out_hbm.at[cid, pl.ds(stripe, rows_per_tile)])

    return k(m, dst, zeros)


def _matmul_bias(x, w, b):
    n, _ = x.shape
    dout = w.shape[1]

    def body(x_ref, w_ref, b_ref, o_ref):
        o_ref[...] = (
            jnp.dot(x_ref[...], w_ref[...], preferred_element_type=jnp.float32)
            + b_ref[...]
        )

    return pl.pallas_call(
        body, out_shape=jax.ShapeDtypeStruct((n, dout), jnp.float32)
    )(x, w, b.reshape(1, dout))


def _edge_messages(edge_attr, hsrc, mlp_w, mlp_b):
    n_edges, ed = edge_attr.shape
    hh = HID * HID
    blk = 8000
    grid = (n_edges // blk,)

    ea_p = edge_attr.reshape(n_edges // 8, 8 * ed)
    hs_p = hsrc.reshape(n_edges // 8, 8 * HID)
    eye8 = jnp.eye(8, dtype=jnp.float32)
    w_big = jnp.kron(eye8, mlp_w)
    b_big = jnp.tile(mlp_b, 8).reshape(1, 8 * hh)
    r_small = (jnp.arange(hh)[None, :] // HID
               == jnp.arange(HID)[:, None]).astype(jnp.float32)
    s_small = (jnp.arange(hh)[:, None] % HID
               == jnp.arange(HID)[None, :]).astype(jnp.float32)
    r_big = jnp.kron(eye8, r_small).astype(jnp.bfloat16)
    s_big = jnp.kron(eye8, s_small).astype(jnp.bfloat16)

    rblk = blk // 8

    def body(ea_ref, hs_ref, w_ref, b_ref, rep_ref, red_ref, m_ref):
        we = jnp.maximum(
            jnp.dot(ea_ref[...], w_ref[...], preferred_element_type=jnp.float32)
            + b_ref[...],
            0.0,
        )
        we16 = we.astype(jnp.bfloat16)
        hs16 = hs_ref[...].astype(jnp.bfloat16)
        rep = jnp.dot(
            hs16, rep_ref[...], preferred_element_type=jnp.float32,
        )
        prod = we16.astype(jnp.float32) * rep
        p_hi = prod.astype(jnp.bfloat16)
        p_lo = (prod - p_hi.astype(jnp.float32)).astype(jnp.bfloat16)
        m_ref[...] = (
            jnp.dot(p_hi, red_ref[...], preferred_element_type=jnp.float32)
            + jnp.dot(p_lo, red_ref[...], preferred_element_type=jnp.float32)
        )

    m_p = pl.pallas_call(
        body,
        grid=grid,
        in_specs=[
            pl.BlockSpec((rblk, 8 * ed), lambda i: (i, 0)),
            pl.BlockSpec((rblk, 8 * HID), lambda i: (i, 0)),
            pl.BlockSpec((8 * ed, 8 * hh), lambda i: (0, 0)),
            pl.BlockSpec((1, 8 * hh), lambda i: (0, 0)),
            pl.BlockSpec((8 * HID, 8 * hh), lambda i: (0, 0)),
            pl.BlockSpec((8 * hh, 8 * HID), lambda i: (0, 0)),
        ],
        out_specs=pl.BlockSpec((rblk, 8 * HID), lambda i: (i, 0)),
        out_shape=jax.ShapeDtypeStruct((n_edges // 8, 8 * HID), jnp.float32),
    )(ea_p, hs_p, w_big, b_big, r_big, s_big)
    return m_p.reshape(n_edges, HID)


def _node_tail(part0, part1, h, lp, bp):
    n, d = h.shape
    trans = bp["final_W"].shape[1]

    def body(p0_ref, p1_ref, h_ref, rootw_ref, convb_ref, gnw_ref, gnb_ref,
             gnms_ref, tw_ref, tb_ref, fw_ref, fb_ref, gfw_ref, gfb_ref,
             gfms_ref, o_ref):
        h_ = h_ref[...]
        out = (
            p0_ref[...] + p1_ref[...]
            + jnp.dot(h_, rootw_ref[...], preferred_element_type=jnp.float32)
            + convb_ref[...]
        )
        mean = jnp.mean(out, axis=0, keepdims=True)
        cent = out - gnms_ref[...] * mean
        var = jnp.mean(cent * cent, axis=0, keepdims=True)
        gn = gnw_ref[...] * cent / jnp.sqrt(var + EPS) + gnb_ref[...]
        h_conv = jnp.maximum(gn, 0.0) + h_
        tw = tw_ref[...]
        h2 = jnp.maximum(
            jnp.dot(h_, tw[:d], preferred_element_type=jnp.float32)
            + jnp.dot(h_conv, tw[d:], preferred_element_type=jnp.float32)
            + tb_ref[...],
            0.0,
        )
        hf = (
            jnp.dot(h2, fw_ref[...], preferred_element_type=jnp.float32)
            + fb_ref[...]
        )
        mean2 = jnp.mean(hf, axis=0, keepdims=True)
        cent2 = hf - gfms_ref[...] * mean2
        var2 = jnp.mean(cent2 * cent2, axis=0, keepdims=True)
        o_ref[...] = jnp.maximum(
            gfw_ref[...] * cent2 / jnp.sqrt(var2 + EPS) + gfb_ref[...], 0.0
        )

    return pl.pallas_call(
        body, out_shape=jax.ShapeDtypeStruct((n, trans), jnp.float32)
    )(
        part0, part1, h,
        lp["root_W"], lp["conv_b"].reshape(1, d),
        lp["gn_w"].reshape(1, d), lp["gn_b"].reshape(1, d),
        lp["gn_ms"].reshape(1, d),
        lp["trans_W"], lp["trans_b"].reshape(1, d),
        bp["final_W"], bp["final_b"].reshape(1, trans),
        bp["gnf_w"].reshape(1, trans), bp["gnf_b"].reshape(1, trans),
        bp["gnf_ms"].reshape(1, trans),
    )


def _jk_combine(h1, h2, jk):
    n, _ = h1.shape
    out_d = jk["projs"][0]["W"].shape[1]

    def body(h1_ref, h2_ref, w1_ref, b1_ref, w2_ref, b2_ref, a1_ref, a2_ref,
             o_ref):
        z1 = (
            jnp.dot(h1_ref[...], w1_ref[...], preferred_element_type=jnp.float32)
            + b1_ref[...]
        )
        z2 = (
            jnp.dot(h2_ref[...], w2_ref[...], preferred_element_type=jnp.float32)
            + b2_ref[...]
        )
        s1 = jnp.dot(z1, a1_ref[...], preferred_element_type=jnp.float32)
        s2 = jnp.dot(z2, a2_ref[...], preferred_element_type=jnp.float32)
        mx = jnp.maximum(s1, s2)
        e1 = jnp.exp(s1 - mx)
        e2 = jnp.exp(s2 - mx)
        o_ref[...] = (e1 * z1 + e2 * z2) / (e1 + e2)

    return pl.pallas_call(
        body, out_shape=jax.ShapeDtypeStruct((n, out_d), jnp.float32)
    )(
        h1, h2,
        jk["projs"][0]["W"], jk["projs"][0]["b"].reshape(1, out_d),
        jk["projs"][1]["W"], jk["projs"][1]["b"].reshape(1, out_d),
        jk["attn"][0].reshape(out_d, 1), jk["attn"][1].reshape(out_d, 1),
    )


def kernel(x, edge_attr, params, edge_index):
    src = edge_index[0]
    dst = edge_index[1]
    n_nodes = x.shape[0]
    hs = []
    h_in = x
    for bp in params["blocks"]:
        h = _matmul_bias(h_in, bp["init_W"], bp["init_b"])
        for lp in bp["layers"]:
            hsrc = _sc_gather_rows(h, src)
            m = _edge_messages(edge_attr, hsrc, lp["mlp_W"], lp["mlp_b"])
            parts = _sc_scatter_add(m, dst, n_nodes)
            h = _node_tail(parts[0], parts[1], h, lp, bp)
        hs.append(h)
        h_in = h
    return _jk_combine(hs[0], hs[1], params["jk"])


# speedup vs baseline: 2.2526x; 1.0715x over previous
import functools

import jax
import jax.numpy as jnp
from jax import lax
from jax.experimental import pallas as pl
from jax.experimental.pallas import tpu as pltpu
from jax.experimental.pallas import tpu_sc as plsc

EPS = 1e-5
HID = 16


def _sc_gather_rows(h, src):
    n_nodes, d = h.shape
    n_edges = src.shape[0]
    info = plsc.get_sparse_core_info()
    nw = info.num_cores * info.num_subcores
    per_w = n_edges // nw
    ch = 2000
    n_ch = per_w // ch
    assert per_w % ch == 0 and n_edges % nw == 0

    mesh = plsc.VectorSubcoreMesh(core_axis_name="c", subcore_axis_name="s")

    @functools.partial(
        pl.kernel,
        out_type=jax.ShapeDtypeStruct((n_edges, d), jnp.float32),
        mesh=mesh,
        compiler_params=pltpu.CompilerParams(use_tc_tiling_on_sc=False),
        scratch_types=[
            pltpu.VMEM((ch,), jnp.int32),
            pltpu.VMEM((ch, d), jnp.float32),
            pltpu.SemaphoreType.DMA,
        ],
    )
    def k(h_hbm, src_hbm, out_hbm, idx_v, rows_v, sem):
        wid = lax.axis_index("s") * info.num_cores + lax.axis_index("c")
        base = wid * per_w

        def body(i, _):
            off = base + i * ch
            pltpu.sync_copy(src_hbm.at[pl.ds(off, ch)], idx_v)
            pltpu.async_copy(h_hbm.at[idx_v], rows_v, sem).wait()
            pltpu.sync_copy(rows_v, out_hbm.at[pl.ds(off, ch)])
            return 0

        lax.fori_loop(0, n_ch, body, 0)

    return k(h, src)


def _sc_scatter_add(m, dst, n_nodes):
    n_edges, d = m.shape
    info = plsc.get_sparse_core_info()
    nc, ns = info.num_cores, info.num_subcores
    nw = nc * ns
    per_w = n_edges // nw
    ch = 2000
    n_ch = per_w // ch
    rows_per_tile = n_nodes // ns
    assert n_nodes % ns == 0 and per_w % ch == 0

    zeros = jnp.zeros((n_nodes, d), jnp.float32)
    mesh = plsc.VectorSubcoreMesh(core_axis_name="c", subcore_axis_name="s")

    @functools.partial(
        pl.kernel,
        out_type=jax.ShapeDtypeStruct((nc, n_nodes, d), jnp.float32),
        mesh=mesh,
        compiler_params=pltpu.CompilerParams(use_tc_tiling_on_sc=False),
        scratch_types=[
            pltpu.VMEM((ch,), jnp.int32),
            pltpu.VMEM((ch, d), jnp.float32),
            pltpu.VMEM_SHARED((n_nodes, d), jnp.float32),
        ],
    )
    def k(m_hbm, dst_hbm, z_hbm, out_hbm, idx_v, rows_v, agg_sh):
        cid = lax.axis_index("c")
        sid = lax.axis_index("s")
        stripe = sid * rows_per_tile
        pltpu.sync_copy(z_hbm.at[pl.ds(stripe, rows_per_tile)],
                        agg_sh.at[pl.ds(stripe, rows_per_tile)])
        plsc.subcore_barrier()

        wid = sid * nc + cid
        base = wid * per_w

        def body(i, _):
            off = base + i * ch
            pltpu.sync_copy(dst_hbm.at[pl.ds(off, ch)], idx_v)
            pltpu.sync_copy(m_hbm.at[pl.ds(off, ch)], rows_v)
            pltpu.sync_copy(rows_v, agg_sh.at[idx_v], add=True)
            return 0

        lax.fori_loop(0, n_ch, body, 0)
        plsc.subcore_barrier()
        pltpu.sync_copy(agg_sh.at[pl.ds(stripe, rows_per_tile)],
                        out_hbm.at[cid, pl.ds(stripe, rows_per_tile)])

    return k(m, dst, zeros)


def _matmul_bias(x, w, b):
    n, _ = x.shape
    dout = w.shape[1]

    def body(x_ref, w_ref, b_ref, o_ref):
        o_ref[...] = (
            jnp.dot(x_ref[...], w_ref[...], preferred_element_type=jnp.float32)
            + b_ref[...]
        )

    return pl.pallas_call(
        body, out_shape=jax.ShapeDtypeStruct((n, dout), jnp.float32)
    )(x, w, b.reshape(1, dout))


def _edge_messages(edge_attr, hsrc, mlp_w, mlp_b):
    n_edges, ed = edge_attr.shape
    hh = HID * HID
    blk = 8000
    grid = (n_edges // blk,)

    ea_p = (edge_attr.T.reshape(ed, n_edges // 8, 8)
            .transpose(1, 0, 2).reshape(n_edges // 8, 8 * ed))
    hs_p = hsrc.reshape(n_edges // 8, 8 * HID)
    eye8 = jnp.eye(8, dtype=jnp.float32)
    w_big = jnp.kron(mlp_w.reshape(ed, 1, hh), jnp.eye(8, dtype=jnp.float32)[:, :, None]).reshape(8 * ed, 8 * hh)
    b_big = jnp.tile(mlp_b, 8).reshape(1, 8 * hh)
    r_small = (jnp.arange(hh)[None, :] // HID
               == jnp.arange(HID)[:, None]).astype(jnp.float32)
    s_small = (jnp.arange(hh)[:, None] % HID
               == jnp.arange(HID)[None, :]).astype(jnp.float32)
    r_big = jnp.kron(eye8, r_small).astype(jnp.bfloat16)
    s_big = jnp.kron(eye8, s_small).astype(jnp.bfloat16)

    rblk = blk // 8

    def body(ea_ref, hs_ref, w_ref, b_ref, rep_ref, red_ref, m_ref):
        we = jnp.maximum(
            jnp.dot(ea_ref[...], w_ref[...], preferred_element_type=jnp.float32)
            + b_ref[...],
            0.0,
        )
        we16 = we.astype(jnp.bfloat16)
        hs16 = hs_ref[...].astype(jnp.bfloat16)
        rep = jnp.dot(
            hs16, rep_ref[...], preferred_element_type=jnp.float32,
        )
        prod = we16.astype(jnp.float32) * rep
        p_hi = prod.astype(jnp.bfloat16)
        p_lo = (prod - p_hi.astype(jnp.float32)).astype(jnp.bfloat16)
        m_ref[...] = (
            jnp.dot(p_hi, red_ref[...], preferred_element_type=jnp.float32)
            + jnp.dot(p_lo, red_ref[...], preferred_element_type=jnp.float32)
        )

    m_p = pl.pallas_call(
        body,
        grid=grid,
        in_specs=[
            pl.BlockSpec((rblk, 8 * ed), lambda i: (i, 0)),
            pl.BlockSpec((rblk, 8 * HID), lambda i: (i, 0)),
            pl.BlockSpec((8 * ed, 8 * hh), lambda i: (0, 0)),
            pl.BlockSpec((1, 8 * hh), lambda i: (0, 0)),
            pl.BlockSpec((8 * HID, 8 * hh), lambda i: (0, 0)),
            pl.BlockSpec((8 * hh, 8 * HID), lambda i: (0, 0)),
        ],
        out_specs=pl.BlockSpec((rblk, 8 * HID), lambda i: (i, 0)),
        out_shape=jax.ShapeDtypeStruct((n_edges // 8, 8 * HID), jnp.float32),
    )(ea_p, hs_p, w_big, b_big, r_big, s_big)
    return m_p.reshape(n_edges, HID)


def _node_tail(part0, part1, h, lp, bp):
    n, d = h.shape
    trans = bp["final_W"].shape[1]

    def body(p0_ref, p1_ref, h_ref, rootw_ref, convb_ref, gnw_ref, gnb_ref,
             gnms_ref, tw_ref, tb_ref, fw_ref, fb_ref, gfw_ref, gfb_ref,
             gfms_ref, o_ref):
        h_ = h_ref[...]
        out = (
            p0_ref[...] + p1_ref[...]
            + jnp.dot(h_, rootw_ref[...], preferred_element_type=jnp.float32)
            + convb_ref[...]
        )
        mean = jnp.mean(out, axis=0, keepdims=True)
        cent = out - gnms_ref[...] * mean
        var = jnp.mean(cent * cent, axis=0, keepdims=True)
        gn = gnw_ref[...] * cent / jnp.sqrt(var + EPS) + gnb_ref[...]
        h_conv = jnp.maximum(gn, 0.0) + h_
        tw = tw_ref[...]
        h2 = jnp.maximum(
            jnp.dot(h_, tw[:d], preferred_element_type=jnp.float32)
            + jnp.dot(h_conv, tw[d:], preferred_element_type=jnp.float32)
            + tb_ref[...],
            0.0,
        )
        hf = (
            jnp.dot(h2, fw_ref[...], preferred_element_type=jnp.float32)
            + fb_ref[...]
        )
        mean2 = jnp.mean(hf, axis=0, keepdims=True)
        cent2 = hf - gfms_ref[...] * mean2
        var2 = jnp.mean(cent2 * cent2, axis=0, keepdims=True)
        o_ref[...] = jnp.maximum(
            gfw_ref[...] * cent2 / jnp.sqrt(var2 + EPS) + gfb_ref[...], 0.0
        )

    return pl.pallas_call(
        body, out_shape=jax.ShapeDtypeStruct((n, trans), jnp.float32)
    )(
        part0, part1, h,
        lp["root_W"], lp["conv_b"].reshape(1, d),
        lp["gn_w"].reshape(1, d), lp["gn_b"].reshape(1, d),
        lp["gn_ms"].reshape(1, d),
        lp["trans_W"], lp["trans_b"].reshape(1, d),
        bp["final_W"], bp["final_b"].reshape(1, trans),
        bp["gnf_w"].reshape(1, trans), bp["gnf_b"].reshape(1, trans),
        bp["gnf_ms"].reshape(1, trans),
    )


def _jk_combine(h1, h2, jk):
    n, _ = h1.shape
    out_d = jk["projs"][0]["W"].shape[1]

    def body(h1_ref, h2_ref, w1_ref, b1_ref, w2_ref, b2_ref, a1_ref, a2_ref,
             o_ref):
        z1 = (
            jnp.dot(h1_ref[...], w1_ref[...], preferred_element_type=jnp.float32)
            + b1_ref[...]
        )
        z2 = (
            jnp.dot(h2_ref[...], w2_ref[...], preferred_element_type=jnp.float32)
            + b2_ref[...]
        )
        s1 = jnp.dot(z1, a1_ref[...], preferred_element_type=jnp.float32)
        s2 = jnp.dot(z2, a2_ref[...], preferred_element_type=jnp.float32)
        mx = jnp.maximum(s1, s2)
        e1 = jnp.exp(s1 - mx)
        e2 = jnp.exp(s2 - mx)
        o_ref[...] = (e1 * z1 + e2 * z2) / (e1 + e2)

    return pl.pallas_call(
        body, out_shape=jax.ShapeDtypeStruct((n, out_d), jnp.float32)
    )(
        h1, h2,
        jk["projs"][0]["W"], jk["projs"][0]["b"].reshape(1, out_d),
        jk["projs"][1]["W"], jk["projs"][1]["b"].reshape(1, out_d),
        jk["attn"][0].reshape(out_d, 1), jk["attn"][1].reshape(out_d, 1),
    )


def kernel(x, edge_attr, params, edge_index):
    src = edge_index[0]
    dst = edge_index[1]
    n_nodes = x.shape[0]
    hs = []
    h_in = x
    for bp in params["blocks"]:
        h = _matmul_bias(h_in, bp["init_W"], bp["init_b"])
        for lp in bp["layers"]:
            hsrc = _sc_gather_rows(h, src)
            m = _edge_messages(edge_attr, hsrc, lp["mlp_W"], lp["mlp_b"])
            parts = _sc_scatter_add(m, dst, n_nodes)
            h = _node_tail(parts[0], parts[1], h, lp, bp)
        hs.append(h)
        h_in = h
    return _jk_combine(hs[0], hs[1], params["jk"])
